# Initial kernel scaffold; baseline (speedup 1.0000x reference)
import jax
import jax.numpy as jnp
from jax.experimental import pallas as pl


def kernel(inputs, patch_h, patch_w, gate_W, gate_b, expert_W, expert_b):
    raise NotImplementedError("write your pallas kernel here")


# speedup vs baseline: 1.0284x; 1.0284x over previous
import functools

import jax
import jax.numpy as jnp
from jax.experimental import pallas as pl
from jax.experimental.pallas import tpu as pltpu

THRESH = 0.125


def _moe_body(x_ref, gw_ref, gb_ref, ew_ref, eb_ref, o_ref, w_scr):
    e = pl.program_id(1)

    @pl.when(e == 0)
    def _init():
        logits = jnp.dot(x_ref[...], gw_ref[...],
                         preferred_element_type=jnp.float32) + gb_ref[...]
        probs = jax.nn.softmax(logits, axis=-1)
        w = jnp.where(probs >= THRESH, probs, 0.0)
        s = jnp.sum(w, axis=-1, keepdims=True)
        s = jnp.where(s == 0.0, 1.0, s)
        w_scr[...] = w / s
        o_ref[...] = jnp.zeros_like(o_ref)

    ncols = w_scr.shape[1]
    lane = jax.lax.broadcasted_iota(jnp.int32, w_scr.shape, 1)
    w_col = jnp.sum(jnp.where(lane == e, w_scr[...], 0.0), axis=1,
                    keepdims=True)
    y = jnp.dot(x_ref[...], ew_ref[0], preferred_element_type=jnp.float32)
    o_ref[...] += w_col * (y + eb_ref[0])


@functools.partial(jax.jit, static_argnums=())
def _moe(x, gate_W, gate_b2, expert_W, expert_b):
    T, D = x.shape
    E = gate_W.shape[-1]
    BT = 1024
    grid = (T // BT, E)
    return pl.pallas_call(
        _moe_body,
        grid=grid,
        in_specs=[
            pl.BlockSpec((BT, D), lambda t, e: (t, 0)),
            pl.BlockSpec((D, E), lambda t, e: (0, 0)),
            pl.BlockSpec((1, E), lambda t, e: (0, 0)),
            pl.BlockSpec((1, D, D), lambda t, e: (e, 0, 0)),
            pl.BlockSpec((1, 1, D), lambda t, e: (e, 0, 0)),
        ],
        out_specs=pl.BlockSpec((BT, D), lambda t, e: (t, 0)),
        out_shape=jax.ShapeDtypeStruct((T, D), jnp.float32),
        scratch_shapes=[pltpu.VMEM((BT, E), jnp.float32)],
    )(x, gate_W, gate_b2, expert_W, expert_b.reshape(E, 1, D))


def kernel(inputs, patch_h, patch_w, gate_W, gate_b, expert_W, expert_b):
    x = inputs.reshape((-1, inputs.shape[-1]))
    out = _moe(x, gate_W, gate_b.reshape(1, -1), expert_W, expert_b)
    return out.reshape(inputs.shape[:-1] + (out.shape[-1],))
